# baseline (speedup 1.0000x reference)
import functools

import jax
import jax.numpy as jnp
from jax import lax
from jax.experimental import pallas as pl
from jax.experimental.pallas import tpu as pltpu
from jax.experimental.pallas import tpu_sc as plsc

NC = 2
NS = 16
L = 16
NW = NC * NS

B = 16384
D = 32
NT = 31250
GC = 16
BPW = B // NW
NSTEP = BPW // GC

_mesh = plsc.VectorSubcoreMesh(core_axis_name="c", subcore_axis_name="s")


@functools.partial(
    pl.kernel,
    mesh=_mesh,
    compiler_params=pltpu.CompilerParams(
        needs_layout_passes=False, use_tc_tiling_on_sc=True),
    out_type=jax.ShapeDtypeStruct((B,), jnp.float32),
    scratch_types=[
        pltpu.VMEM((BPW,), jnp.int32),
        pltpu.VMEM((BPW,), jnp.int32),
        pltpu.VMEM((NSTEP, GC), jnp.int32),
        pltpu.VMEM((NSTEP, GC), jnp.int32),
        pltpu.VMEM((BPW,), jnp.int32),
        pltpu.VMEM((BPW,), jnp.int32),
        pltpu.VMEM((BPW,), jnp.int32),
        pltpu.VMEM((BPW,), jnp.int32),
        pltpu.VMEM((2, GC, 8, 128), jnp.float32),
        pltpu.VMEM((2, GC, 8, 128), jnp.float32),
        pltpu.VMEM((BPW,), jnp.float32),
        pltpu.SemaphoreType.DMA,
        pltpu.SemaphoreType.DMA,
    ],
)
def _sc_embed_dot(x0_hbm, x1_hbm, tab_hbm, out_hbm,
                  x0_v, x1_v, tid0_v, tid1_v, sub0_v, sub1_v, col0_v, col1_v,
                  tiles0_v, tiles1_v, out_v, sem_a, sem_b):
    wid = lax.axis_index("s") * NC + lax.axis_index("c")
    base = wid * BPW

    pltpu.sync_copy(x0_hbm.at[pl.ds(base, BPW)], x0_v)
    pltpu.sync_copy(x1_hbm.at[pl.ds(base, BPW)], x1_v)

    for g in range(BPW // L):
        a = x0_v[pl.ds(g * L, L)]
        b = a + x1_v[pl.ds(g * L, L)]
        tid0_v[g, pl.ds(0, L)] = a >> 5
        tid1_v[g, pl.ds(0, L)] = b >> 5
        sub0_v[pl.ds(g * L, L)] = (a >> 2) & 7
        sub1_v[pl.ds(g * L, L)] = (b >> 2) & 7
        col0_v[pl.ds(g * L, L)] = (a & 3) * D
        col1_v[pl.ds(g * L, L)] = (b & 3) * D

    sems = (sem_a, sem_b)
    lanes = lax.iota(jnp.int32, L)

    def fire(step, slot):
        s = sems[slot]
        pltpu.async_copy(tab_hbm.at[tid0_v.at[step]], tiles0_v.at[slot], s)
        pltpu.async_copy(tab_hbm.at[tid1_v.at[step]], tiles1_v.at[slot], s)

    def drain(slot):
        pltpu.make_async_copy(
            tab_hbm.at[tid0_v.at[0]], tiles0_v.at[slot], sems[slot]).wait()
        pltpu.make_async_copy(
            tab_hbm.at[tid1_v.at[0]], tiles1_v.at[slot], sems[slot]).wait()

    def compute(step, slot):
        s0v = sub0_v[pl.ds(step * GC, GC)]
        s1v = sub1_v[pl.ds(step * GC, GC)]
        c0v = col0_v[pl.ds(step * GC, GC)]
        c1v = col1_v[pl.ds(step * GC, GC)]
        acc = jnp.zeros((L,), jnp.float32)
        for k in range(GC):
            s0 = s0v[k]
            s1 = s1v[k]
            c0 = c0v[k]
            c1 = c1v[k]
            a0 = tiles0_v[slot, k, s0, pl.ds(c0, L)]
            a1 = tiles0_v[slot, k, s0, pl.ds(c0 + L, L)]
            b0 = tiles1_v[slot, k, s1, pl.ds(c1, L)]
            b1 = tiles1_v[slot, k, s1, pl.ds(c1 + L, L)]
            s = jnp.sum(a0 * b0 + a1 * b1)
            acc = jnp.where(lanes == k, s, acc)
        out_v[pl.ds(step * GC, GC)] = acc

    fire(0, 0)
    fire(1, 1)

    def pair_body(j, _):
        step0 = 2 * j
        drain(0)
        compute(step0, 0)

        @pl.when(j < NSTEP // 2 - 1)
        def _():
            fire(step0 + 2, 0)

        drain(1)
        compute(step0 + 1, 1)

        @pl.when(j < NSTEP // 2 - 1)
        def _():
            fire(step0 + 3, 1)

        return 0
    lax.fori_loop(0, NSTEP // 2, pair_body, 0)

    pltpu.sync_copy(out_v, out_hbm.at[pl.ds(base, BPW)])


def kernel(x, table):
    x0 = x[:, 0]
    x1 = x[:, 1]
    t3 = table.reshape(NT, 8, 128)
    return _sc_embed_dot(x0, x1, t3)


# speedup vs baseline: 1.7728x; 1.7728x over previous
import functools

import jax
import jax.numpy as jnp
from jax import lax
from jax.experimental import pallas as pl
from jax.experimental.pallas import tpu as pltpu
from jax.experimental.pallas import tpu_sc as plsc

NC = 2
NS = 16
L = 16
NW = NC * NS

B = 16384
D = 32
GC = 16
BPW = B // NW
NSTEP = BPW // GC

_mesh = plsc.VectorSubcoreMesh(core_axis_name="c", subcore_axis_name="s")


@functools.partial(
    pl.kernel,
    mesh=_mesh,
    compiler_params=pltpu.CompilerParams(
        needs_layout_passes=False, use_tc_tiling_on_sc=True),
    out_type=jax.ShapeDtypeStruct((B,), jnp.float32),
    scratch_types=[
        pltpu.VMEM((BPW,), jnp.int32),
        pltpu.VMEM((BPW,), jnp.int32),
        pltpu.VMEM((2, GC, 1, D), jnp.float32),
        pltpu.VMEM((2, GC, 1, D), jnp.float32),
        pltpu.VMEM((BPW,), jnp.float32),
        pltpu.SemaphoreType.DMA,
        pltpu.SemaphoreType.DMA,
    ],
)
def _sc_embed_dot(x0_hbm, x1_hbm, tab_hbm, out_hbm,
                  i0_v, i1_v, rows0_v, rows1_v, out_v, sem_a, sem_b):
    wid = lax.axis_index("s") * NC + lax.axis_index("c")
    base = wid * BPW

    pltpu.sync_copy(x0_hbm.at[pl.ds(base, BPW)], i0_v)
    pltpu.sync_copy(x1_hbm.at[pl.ds(base, BPW)], i1_v)

    for g in range(BPW // L):
        a = i0_v[pl.ds(g * L, L)]
        b = i1_v[pl.ds(g * L, L)]
        i1_v[pl.ds(g * L, L)] = a + b

    sems = (sem_a, sem_b)
    lanes = lax.iota(jnp.int32, L)

    def fire(step, slot):
        s = sems[slot]
        iv0 = i0_v[pl.ds(step * GC, GC)]
        iv1 = i1_v[pl.ds(step * GC, GC)]
        for k in range(GC):
            pltpu.async_copy(
                tab_hbm.at[pl.ds(iv0[k], 1), :], rows0_v.at[slot, k], s)
            pltpu.async_copy(
                tab_hbm.at[pl.ds(iv1[k], 1), :], rows1_v.at[slot, k], s)

    def drain(slot):
        for k in range(GC):
            pltpu.make_async_copy(
                tab_hbm.at[pl.ds(0, 1), :], rows0_v.at[slot, k], sems[slot]
            ).wait()
            pltpu.make_async_copy(
                tab_hbm.at[pl.ds(0, 1), :], rows1_v.at[slot, k], sems[slot]
            ).wait()

    def compute(step, slot):
        acc = jnp.zeros((L,), jnp.float32)
        for k in range(GC):
            a0 = rows0_v[slot, k, 0, pl.ds(0, L)]
            a1 = rows0_v[slot, k, 0, pl.ds(L, L)]
            b0 = rows1_v[slot, k, 0, pl.ds(0, L)]
            b1 = rows1_v[slot, k, 0, pl.ds(L, L)]
            s = jnp.sum(a0 * b0 + a1 * b1)
            acc = jnp.where(lanes == k, s, acc)
        out_v[pl.ds(step * GC, GC)] = acc

    fire(0, 0)
    fire(1, 1)

    def pair_body(j, _):
        step0 = 2 * j
        drain(0)
        compute(step0, 0)

        @pl.when(j < NSTEP // 2 - 1)
        def _():
            fire(step0 + 2, 0)

        drain(1)
        compute(step0 + 1, 1)

        @pl.when(j < NSTEP // 2 - 1)
        def _():
            fire(step0 + 3, 1)

        return 0
    lax.fori_loop(0, NSTEP // 2, pair_body, 0)

    pltpu.sync_copy(out_v, out_hbm.at[pl.ds(base, BPW)])


def kernel(x, table):
    x0 = x[:, 0]
    x1 = x[:, 1]
    return _sc_embed_dot(x0, x1, table)


# speedup vs baseline: 2.9217x; 1.6481x over previous
import functools

import jax
import jax.numpy as jnp
from jax import lax
from jax.experimental import pallas as pl
from jax.experimental.pallas import tpu as pltpu
from jax.experimental.pallas import tpu_sc as plsc

NC = 2
NS = 16
L = 16
NW = NC * NS

B = 16384
D = 32
GC = 16
BPW = B // NW
NSTEP = BPW // GC

_mesh = plsc.VectorSubcoreMesh(core_axis_name="c", subcore_axis_name="s")


@functools.partial(
    pl.kernel,
    mesh=_mesh,
    compiler_params=pltpu.CompilerParams(
        needs_layout_passes=False, use_tc_tiling_on_sc=True),
    out_type=jax.ShapeDtypeStruct((B,), jnp.float32),
    scratch_types=[
        pltpu.VMEM((BPW,), jnp.int32),
        pltpu.VMEM((BPW,), jnp.int32),
        pltpu.VMEM((2, GC, 1, D), jnp.float32),
        pltpu.VMEM((2, GC, 1, D), jnp.float32),
        pltpu.VMEM((BPW,), jnp.float32),
        pltpu.SemaphoreType.DMA,
        pltpu.SemaphoreType.DMA,
    ],
)
def _sc_embed_dot(x0_hbm, x1_hbm, tab3_hbm, out_hbm,
                  i0_v, i1_v, rows0_v, rows1_v, out_v, sem_a, sem_b):
    wid = lax.axis_index("s") * NC + lax.axis_index("c")
    base = wid * BPW
    tab_hbm = tab3_hbm.at[0]

    pltpu.sync_copy(x0_hbm.at[pl.ds(base, BPW)], i0_v)
    pltpu.sync_copy(x1_hbm.at[pl.ds(base, BPW)], i1_v)

    for g in range(BPW // L):
        a = i0_v[pl.ds(g * L, L)]
        b = i1_v[pl.ds(g * L, L)]
        i1_v[pl.ds(g * L, L)] = a + b

    sems = (sem_a, sem_b)
    lanes = lax.iota(jnp.int32, L)

    def fire(step, slot):
        s = sems[slot]
        iv0 = i0_v[pl.ds(step * GC, GC)]
        iv1 = i1_v[pl.ds(step * GC, GC)]
        for k in range(GC):
            pltpu.async_copy(
                tab_hbm.at[pl.ds(iv0[k], 1), :], rows0_v.at[slot, k], s)
            pltpu.async_copy(
                tab_hbm.at[pl.ds(iv1[k], 1), :], rows1_v.at[slot, k], s)

    def drain(slot):
        for k in range(GC):
            pltpu.make_async_copy(
                tab_hbm.at[pl.ds(0, 1), :], rows0_v.at[slot, k], sems[slot]
            ).wait()
            pltpu.make_async_copy(
                tab_hbm.at[pl.ds(0, 1), :], rows1_v.at[slot, k], sems[slot]
            ).wait()

    def compute(step, slot):
        acc = jnp.zeros((L,), jnp.float32)
        for k in range(GC):
            a0 = rows0_v[slot, k, 0, pl.ds(0, L)]
            a1 = rows0_v[slot, k, 0, pl.ds(L, L)]
            b0 = rows1_v[slot, k, 0, pl.ds(0, L)]
            b1 = rows1_v[slot, k, 0, pl.ds(L, L)]
            s = jnp.sum(a0 * b0 + a1 * b1)
            acc = jnp.where(lanes == k, s, acc)
        out_v[pl.ds(step * GC, GC)] = acc

    fire(0, 0)
    fire(1, 1)

    def pair_body(j, _):
        step0 = 2 * j
        drain(0)
        compute(step0, 0)

        @pl.when(j < NSTEP // 2 - 1)
        def _():
            fire(step0 + 2, 0)

        drain(1)
        compute(step0 + 1, 1)

        @pl.when(j < NSTEP // 2 - 1)
        def _():
            fire(step0 + 3, 1)

        return 0
    lax.fori_loop(0, NSTEP // 2, pair_body, 0)

    pltpu.sync_copy(out_v, out_hbm.at[pl.ds(base, BPW)])


def kernel(x, table):
    x0 = x[:, 0]
    x1 = x[:, 1]
    return _sc_embed_dot(x0, x1, table.reshape(1, 1000000, D))


# speedup vs baseline: 2.9293x; 1.0026x over previous
import functools

import jax
import jax.numpy as jnp
from jax import lax
from jax.experimental import pallas as pl
from jax.experimental.pallas import tpu as pltpu
from jax.experimental.pallas import tpu_sc as plsc

NC = 2
NS = 16
L = 16
NW = NC * NS

B = 16384
D = 32
GC = 16
BPW = B // NW
NSTEP = BPW // GC

_mesh = plsc.VectorSubcoreMesh(core_axis_name="c", subcore_axis_name="s")


@functools.partial(
    pl.kernel,
    mesh=_mesh,
    compiler_params=pltpu.CompilerParams(
        needs_layout_passes=False, use_tc_tiling_on_sc=True),
    out_type=jax.ShapeDtypeStruct((B,), jnp.float32),
    scratch_types=[
        pltpu.VMEM((BPW,), jnp.int32),
        pltpu.VMEM((BPW,), jnp.int32),
        pltpu.VMEM((2, GC, 1, D), jnp.float32),
        pltpu.VMEM((2, GC, 1, D), jnp.float32),
        pltpu.VMEM((2 * GC, D), jnp.float32),
        pltpu.VMEM((BPW,), jnp.float32),
        pltpu.SemaphoreType.DMA,
        pltpu.SemaphoreType.DMA,
    ],
)
def _sc_embed_dot(x0_hbm, x1_hbm, tab3_hbm, out_hbm,
                  i0_v, i1_v, rows0_v, rows1_v, drain_v, out_v, sem_a, sem_b):
    wid = lax.axis_index("s") * NC + lax.axis_index("c")
    base = wid * BPW
    tab_hbm = tab3_hbm.at[0]

    pltpu.sync_copy(x0_hbm.at[pl.ds(base, BPW)], i0_v)
    pltpu.sync_copy(x1_hbm.at[pl.ds(base, BPW)], i1_v)

    for g in range(BPW // L):
        a = i0_v[pl.ds(g * L, L)]
        b = i1_v[pl.ds(g * L, L)]
        i1_v[pl.ds(g * L, L)] = a + b

    sems = (sem_a, sem_b)
    lanes = lax.iota(jnp.int32, L)

    def fire(step, slot):
        s = sems[slot]
        iv0 = i0_v[pl.ds(step * GC, GC)]
        iv1 = i1_v[pl.ds(step * GC, GC)]
        for k in range(GC):
            pltpu.async_copy(
                tab_hbm.at[pl.ds(iv0[k], 1), :], rows0_v.at[slot, k], s)
            pltpu.async_copy(
                tab_hbm.at[pl.ds(iv1[k], 1), :], rows1_v.at[slot, k], s)

    def drain(slot):
        pltpu.make_async_copy(
            tab_hbm.at[pl.ds(0, 2 * GC), :], drain_v, sems[slot]).wait()

    def compute(step, slot):
        acc = jnp.zeros((L,), jnp.float32)
        for k in range(GC):
            a0 = rows0_v[slot, k, 0, pl.ds(0, L)]
            a1 = rows0_v[slot, k, 0, pl.ds(L, L)]
            b0 = rows1_v[slot, k, 0, pl.ds(0, L)]
            b1 = rows1_v[slot, k, 0, pl.ds(L, L)]
            s = jnp.sum(a0 * b0 + a1 * b1)
            acc = jnp.where(lanes == k, s, acc)
        out_v[pl.ds(step * GC, GC)] = acc

    fire(0, 0)
    fire(1, 1)

    def pair_body(j, _):
        step0 = 2 * j
        drain(0)
        compute(step0, 0)

        @pl.when(j < NSTEP // 2 - 1)
        def _():
            fire(step0 + 2, 0)

        drain(1)
        compute(step0 + 1, 1)

        @pl.when(j < NSTEP // 2 - 1)
        def _():
            fire(step0 + 3, 1)

        return 0
    lax.fori_loop(0, NSTEP // 2, pair_body, 0)

    pltpu.sync_copy(out_v, out_hbm.at[pl.ds(base, BPW)])


def kernel(x, table):
    x0 = x[:, 0]
    x1 = x[:, 1]
    return _sc_embed_dot(x0, x1, table.reshape(1, 1000000, D))
